# Initial kernel scaffold; baseline (speedup 1.0000x reference)
import jax
import jax.numpy as jnp
from jax.experimental import pallas as pl


def kernel(data, query, W, b):
    raise NotImplementedError("write your pallas kernel here")


# speedup vs baseline: 1357.1708x; 1357.1708x over previous
import jax
import jax.numpy as jnp
from jax.experimental import pallas as pl

_NA = 64
_ND = 4096
_NQ = 1024
_RB = 512
_CB = 512


def _rank_body(xt_ref, w_ref, b_ref, out_ref):
    lt = jax.lax.dot_general(
        w_ref[...], xt_ref[...], (((1,), (0,)), ((), ())),
        preferred_element_type=jnp.float32)
    lt = lt + b_ref[...]
    rank = jnp.ones(lt.shape, jnp.float32)
    for a in range(_NA):
        rank += (lt[a:a + 1, :] > lt).astype(jnp.float32)
    out_ref[...] = rank.astype(jnp.bfloat16)


def _mm_body(q_ref, d_ref, out_ref):
    out_ref[...] = jax.lax.dot_general(
        q_ref[...], d_ref[...], (((1,), (0,)), ((), ())),
        preferred_element_type=jnp.float32)


def kernel(data, query, W, b):
    with jax.enable_x64(False):
        return _kernel32(data, query, W, b)


def _kernel32(data, query, W, b):
    rows_t = jnp.concatenate([data, query], axis=0).T
    nrows = _ND + _NQ
    ranks_t = pl.pallas_call(
        _rank_body,
        grid=(nrows // _RB,),
        in_specs=[
            pl.BlockSpec((128, _RB), lambda i: (0, i)),
            pl.BlockSpec((_NA, 128), lambda i: (0, 0)),
            pl.BlockSpec((_NA, 1), lambda i: (0, 0)),
        ],
        out_specs=pl.BlockSpec((_NA, _RB), lambda i: (0, i)),
        out_shape=jax.ShapeDtypeStruct((_NA, nrows), jnp.bfloat16),
    )(rows_t, W, b.reshape(_NA, 1))
    d_rank_t = ranks_t[:, :_ND]
    q_rank = ranks_t[:, _ND:].T
    out = pl.pallas_call(
        _mm_body,
        grid=(_ND // _CB,),
        in_specs=[
            pl.BlockSpec((_NQ, _NA), lambda j: (0, 0)),
            pl.BlockSpec((_NA, _CB), lambda j: (0, j)),
        ],
        out_specs=pl.BlockSpec((_NQ, _CB), lambda j: (0, j)),
        out_shape=jax.ShapeDtypeStruct((_NQ, _ND), jnp.float32),
    )(q_rank, d_rank_t)
    return out
